# baseline (speedup 1.0000x reference)
import functools

import jax
import jax.numpy as jnp
from jax import lax
from jax.experimental import pallas as pl
from jax.experimental.pallas import tpu as pltpu
from jax.experimental.pallas import tpu_sc as plsc

_D = 64
_SCALE = 8.0
_NW = 32
_C = 128
_LANES = 16


@functools.partial(jax.jit, static_argnames=("nch",))
def _emb_lookup(tok, table, nch):
    mesh = plsc.VectorSubcoreMesh(core_axis_name="c", subcore_axis_name="s")

    @functools.partial(
        pl.kernel,
        mesh=mesh,
        out_type=jax.ShapeDtypeStruct((_NW, nch, _C, _D), jnp.float32),
        compiler_params=pltpu.CompilerParams(use_tc_tiling_on_sc=False),
        scratch_types=[
            pltpu.VMEM((nch, _C), jnp.int32),
            pltpu.VMEM((_C, _D), jnp.float32),
            pltpu.VMEM((_C, _D), jnp.float32),
            pltpu.SemaphoreType.DMA,
            pltpu.SemaphoreType.DMA,
        ],
    )
    def body(tok_hbm, table_hbm, out_hbm, idx_v, buf0, buf1, sem0, sem1):
        cid = lax.axis_index("c")
        sid = lax.axis_index("s")
        wid = sid * 2 + cid

        pltpu.sync_copy(tok_hbm.at[wid], idx_v)

        bufs = (buf0, buf1)
        sems = (sem0, sem1)

        def start_gather(chunk, b):
            pltpu.make_async_copy(
                table_hbm.at[idx_v.at[chunk]], bufs[b], sems[b]
            ).start()

        def wait_gather(chunk, b):
            pltpu.make_async_copy(
                table_hbm.at[idx_v.at[chunk]], bufs[b], sems[b]
            ).wait()

        start_gather(0, 0)
        start_gather(1, 1)

        n_vec = _C * _D // _LANES

        def loop_body(i, carry):
            for b in range(2):
                chunk = i * 2 + b
                wait_gather(chunk, b)

                def scale_body(k, c2):
                    r = k >> 2
                    col = (k & 3) * _LANES
                    buf = bufs[b]
                    buf[r, pl.ds(col, _LANES)] = (
                        buf[r, pl.ds(col, _LANES)] * _SCALE
                    )
                    return c2

                lax.fori_loop(0, n_vec, scale_body, 0, unroll=8)
                pltpu.sync_copy(bufs[b], out_hbm.at[wid, chunk])

                @pl.when(chunk + 2 < nch)
                def _():
                    start_gather(chunk + 2, b)
            return carry

        lax.fori_loop(0, nch // 2, loop_body, 0)

    return body(tok, table)


def kernel(tokens, embedding):
    bsz, seq = tokens.shape
    tot = bsz * seq
    nch = tot // (_NW * _C)
    tok = tokens.astype(jnp.int32).reshape(_NW, nch, _C)
    out = _emb_lookup(tok, embedding, nch)
    return out.reshape(bsz, seq, _D)
